# baseline (speedup 1.0000x reference)
import functools

import jax
import jax.numpy as jnp
from jax import lax
from jax.experimental import pallas as pl
from jax.experimental.pallas import tpu as pltpu
from jax.experimental.pallas import tpu_sc as plsc

N = 10000
E = 320000
D_IN = 128
D_HID = 64
D_OUT = 64
NEG_SLOPE = 0.2

NC = 2
NS = 16
NW = NC * NS
L = 16

CH = 128
EPT = E // NW
NFULL = EPT // CH
REM = EPT - NFULL * CH
NB = 3
SB = 48
ZD = 208
ROWS_PT = 624
ROWS_TAIL = N - ROWS_PT * NS
N_PAD = 10240


def _tc_embed_body(x_ref, w_ref, asv_ref, adv_ref, h_ref, s_ref, d_ref):
    h = jnp.dot(x_ref[...], w_ref[...], preferred_element_type=jnp.float32)
    h_ref[...] = h
    s_ref[...] = jnp.dot(h, asv_ref[...], preferred_element_type=jnp.float32)
    d_ref[...] = jnp.dot(h, adv_ref[...], preferred_element_type=jnp.float32)


def _tc_embed(x, W, a_src, a_dst):
    n = x.shape[0]
    dh = W.shape[1]
    h, s, d = pl.pallas_call(
        _tc_embed_body,
        out_shape=[
            jax.ShapeDtypeStruct((n, dh), jnp.float32),
            jax.ShapeDtypeStruct((n, 1), jnp.float32),
            jax.ShapeDtypeStruct((n, 1), jnp.float32),
        ],
    )(x, W, a_src[:, None], a_dst[:, None])
    return h, s[:, 0], d[:, 0]


def _tc_combine_embed_body(p_ref, d_ref, w_ref, asv_ref, adv_ref,
                           h_ref, s_ref, dd_ref):
    denom = d_ref[0] + d_ref[1] + 1e-16
    z = (p_ref[0] + p_ref[1]) / denom
    g = jnp.where(z > 0, z, jnp.exp(z) - 1.0)
    h = jnp.dot(g, w_ref[...], preferred_element_type=jnp.float32)
    h_ref[...] = h
    s_ref[...] = jnp.dot(h, asv_ref[...], preferred_element_type=jnp.float32)
    dd_ref[...] = jnp.dot(h, adv_ref[...], preferred_element_type=jnp.float32)


def _tc_combine_embed(p, d, W, a_src, a_dst):
    n = p.shape[1]
    dh = W.shape[1]
    h, s, dd = pl.pallas_call(
        _tc_combine_embed_body,
        out_shape=[
            jax.ShapeDtypeStruct((n, dh), jnp.float32),
            jax.ShapeDtypeStruct((n, 1), jnp.float32),
            jax.ShapeDtypeStruct((n, 1), jnp.float32),
        ],
    )(p, d[:, :, None], W, a_src[:, None], a_dst[:, None])
    return h, s[:, 0], dd[:, 0]


def _tc_finalize_body(p_ref, d_ref, o_ref):
    denom = d_ref[0] + d_ref[1] + 1e-16
    o_ref[...] = (p_ref[0] + p_ref[1]) / denom


def _tc_finalize(p, d):
    n = p.shape[1]
    dh = p.shape[2]
    return pl.pallas_call(
        _tc_finalize_body,
        out_shape=jax.ShapeDtypeStruct((n, dh), jnp.float32),
    )(p, d[:, :, None])


def _sc_edge_body(h_hbm, src_hbm, dst_hbm, as_hbm, ad_hbm,
                  p_hbm, dn_hbm,
                  h_sp, acc_sp, den_sp,
                  as_v, ad_v, srcq, dstq, w_q, rowsq, rows_v,
                  rsrc, rdst, rw, rrows, zd_v,
                  isem, gsem, ssem):
    cid = lax.axis_index("c")
    sid = lax.axis_index("s")
    wid = sid * NC + cid

    r0 = sid * ROWS_PT
    nsb = ROWS_PT // SB

    pltpu.sync_copy(as_hbm, as_v)
    pltpu.sync_copy(ad_hbm, ad_v)

    def _stage(i, _):
        pltpu.sync_copy(h_hbm.at[pl.ds(r0 + i * SB, SB)], rows_v)
        pltpu.sync_copy(rows_v, h_sp.at[pl.ds(r0 + i * SB, SB)])
        return _
    lax.fori_loop(0, nsb, _stage, None)

    @pl.when(sid == NS - 1)
    def _tail_stage():
        t0 = N - ROWS_TAIL
        pltpu.sync_copy(h_hbm.at[pl.ds(t0, ROWS_TAIL)],
                        rows_v.at[pl.ds(0, ROWS_TAIL)])
        pltpu.sync_copy(rows_v.at[pl.ds(0, ROWS_TAIL)],
                        h_sp.at[pl.ds(t0, ROWS_TAIL)])

    zeros16 = jnp.zeros((L,), jnp.float32)

    def _zrow(i, _):
        for j in range(D_HID // L):
            rows_v[i, pl.ds(j * L, L)] = zeros16
        return _
    lax.fori_loop(0, SB, _zrow, None)

    def _zd(i, _):
        zd_v[pl.ds(i * L, L)] = zeros16
        return _
    lax.fori_loop(0, ZD // L, _zd, None)

    def _zacc(i, _):
        pltpu.sync_copy(rows_v, acc_sp.at[pl.ds(r0 + i * SB, SB)])
        return _
    lax.fori_loop(0, nsb, _zacc, None)

    def _zden(i, _):
        pltpu.sync_copy(zd_v, den_sp.at[pl.ds(r0 + i * ZD, ZD)])
        return _
    lax.fori_loop(0, ROWS_PT // ZD, _zden, None)

    @pl.when(sid == NS - 1)
    def _tail_zero():
        t0 = N - ROWS_TAIL
        pltpu.sync_copy(rows_v.at[pl.ds(0, ROWS_TAIL)],
                        acc_sp.at[pl.ds(t0, ROWS_TAIL)])
        pltpu.sync_copy(zd_v.at[pl.ds(0, ROWS_TAIL)],
                        den_sp.at[pl.ds(t0, ROWS_TAIL)])

    plsc.subcore_barrier()

    e0 = wid * EPT

    def _issue_idx(c, b):
        base = e0 + c * CH
        pltpu.async_copy(src_hbm.at[pl.ds(base, CH)], srcq.at[b], isem.at[b])
        pltpu.async_copy(dst_hbm.at[pl.ds(base, CH)], dstq.at[b], isem.at[b])

    def _wait_idx(c, b):
        base = e0 + c * CH
        pltpu.make_async_copy(
            src_hbm.at[pl.ds(base, CH)], srcq.at[b], isem.at[b]).wait()
        pltpu.make_async_copy(
            dst_hbm.at[pl.ds(base, CH)], dstq.at[b], isem.at[b]).wait()

    def _issue_gather(b):
        pltpu.async_copy(h_sp.at[srcq.at[b]], rowsq.at[b], gsem.at[b])

    def _wait_gather(b):
        pltpu.make_async_copy(
            h_sp.at[srcq.at[b]], rowsq.at[b], gsem.at[b]).wait()

    def _issue_scatter(b):
        pltpu.async_copy(rowsq.at[b], acc_sp.at[dstq.at[b]], ssem.at[b],
                         add=True)
        pltpu.async_copy(w_q.at[b], den_sp.at[dstq.at[b]], ssem.at[b],
                         add=True)

    def _drain_scatter(b):
        pltpu.make_async_copy(
            rowsq.at[b], acc_sp.at[dstq.at[b]], ssem.at[b]).wait()
        pltpu.make_async_copy(
            w_q.at[b], den_sp.at[dstq.at[b]], ssem.at[b]).wait()

    def _compute_w(b):
        @plsc.parallel_loop(0, CH // L, unroll=2)
        def _w(i):
            s16 = srcq[b, pl.ds(i * L, L)]
            d16 = dstq[b, pl.ds(i * L, L)]
            e = plsc.load_gather(as_v, [s16]) + plsc.load_gather(ad_v, [d16])
            e = jnp.where(e > 0, e, NEG_SLOPE * e)
            w_q[b, pl.ds(i * L, L)] = jnp.exp(e)

    def _scale_rows(b):
        @plsc.parallel_loop(0, CH // L, unroll=1)
        def _s(i):
            for t in range(L):
                wv = plsc.load_gather(
                    w_q.at[b], [jnp.full((L,), i * L + t, jnp.int32)])
                for j in range(D_HID // L):
                    sl = pl.ds(j * L, L)
                    rowsq[b, i * L + t, sl] = rowsq[b, i * L + t, sl] * wv

    _issue_idx(0, 0)
    _issue_idx(1, 1)
    _wait_idx(0, 0)
    _issue_gather(0)

    def _chunk(ci, _):
        b = lax.rem(ci, NB)
        s1 = lax.rem(ci + 1, NB)
        s2 = lax.rem(ci + 2, NB)
        _compute_w(b)
        _wait_gather(b)
        _scale_rows(b)
        _issue_scatter(b)

        @pl.when(ci + 1 < NFULL)
        def _prep_gather():
            _wait_idx(ci + 1, s1)
            _issue_gather(s1)

        @pl.when(ci >= NB - 2)
        def _drain_prev():
            _drain_scatter(s2)

        @pl.when(ci + 2 < NFULL)
        def _prep_idx():
            _issue_idx(ci + 2, s2)
        return _

    lax.fori_loop(0, NFULL, _chunk, None)

    for cd in range(NFULL - (NB - 2), NFULL):
        _drain_scatter(jnp.int32(cd % NB))

    rbase = e0 + NFULL * CH
    pltpu.sync_copy(src_hbm.at[pl.ds(rbase, REM)], rsrc)
    pltpu.sync_copy(dst_hbm.at[pl.ds(rbase, REM)], rdst)
    s16 = rsrc[...]
    d16 = rdst[...]
    e = plsc.load_gather(as_v, [s16]) + plsc.load_gather(ad_v, [d16])
    e = jnp.where(e > 0, e, NEG_SLOPE * e)
    rw[...] = jnp.exp(e)
    pltpu.sync_copy(h_sp.at[rsrc], rrows)

    @plsc.parallel_loop(0, REM, unroll=4)
    def _rscale(i):
        wv = plsc.load_gather(rw, [jnp.full((L,), i, jnp.int32)])
        for j in range(D_HID // L):
            sl = pl.ds(j * L, L)
            rrows[i, sl] = rrows[i, sl] * wv

    pltpu.sync_copy(rrows, acc_sp.at[rdst], add=True)
    pltpu.sync_copy(rw, den_sp.at[rdst], add=True)

    plsc.subcore_barrier()

    def _wb(i, _):
        pltpu.sync_copy(acc_sp.at[pl.ds(r0 + i * SB, SB)], rows_v)
        pltpu.sync_copy(rows_v, p_hbm.at[cid, pl.ds(r0 + i * SB, SB)])
        return _
    lax.fori_loop(0, nsb, _wb, None)

    def _wbden(i, _):
        pltpu.sync_copy(den_sp.at[pl.ds(r0 + i * ZD, ZD)], zd_v)
        pltpu.sync_copy(zd_v, dn_hbm.at[pl.ds(cid * N_PAD + r0 + i * ZD, ZD)])
        return _
    lax.fori_loop(0, ROWS_PT // ZD, _wbden, None)

    @pl.when(sid == NS - 1)
    def _tail_out():
        t0 = N - ROWS_TAIL
        pltpu.sync_copy(acc_sp.at[pl.ds(t0, ROWS_TAIL)],
                        rows_v.at[pl.ds(0, ROWS_TAIL)])
        pltpu.sync_copy(rows_v.at[pl.ds(0, ROWS_TAIL)],
                        p_hbm.at[cid, pl.ds(t0, ROWS_TAIL)])
        pltpu.sync_copy(den_sp.at[pl.ds(t0, ROWS_TAIL)],
                        zd_v.at[pl.ds(0, ROWS_TAIL)])
        pltpu.sync_copy(zd_v.at[pl.ds(0, ROWS_TAIL)],
                        dn_hbm.at[pl.ds(cid * N_PAD + t0, ROWS_TAIL)])


_sc_edge = pl.kernel(
    _sc_edge_body,
    out_type=(
        jax.ShapeDtypeStruct((NC, N, D_HID), jnp.float32),
        jax.ShapeDtypeStruct((NC * N_PAD,), jnp.float32),
    ),
    mesh=plsc.VectorSubcoreMesh(
        core_axis_name="c", subcore_axis_name="s",
        num_cores=NC, num_subcores=NS),
    compiler_params=pltpu.CompilerParams(
        needs_layout_passes=False, use_tc_tiling_on_sc=False),
    scratch_types=[
        pltpu.VMEM_SHARED((N, D_HID), jnp.float32),
        pltpu.VMEM_SHARED((N, D_HID), jnp.float32),
        pltpu.VMEM_SHARED((N,), jnp.float32),
        pltpu.VMEM((N,), jnp.float32),
        pltpu.VMEM((N,), jnp.float32),
        pltpu.VMEM((NB, CH), jnp.int32),
        pltpu.VMEM((NB, CH), jnp.int32),
        pltpu.VMEM((NB, CH), jnp.float32),
        pltpu.VMEM((NB, CH, D_HID), jnp.float32),
        pltpu.VMEM((SB, D_HID), jnp.float32),
        pltpu.VMEM((REM,), jnp.int32),
        pltpu.VMEM((REM,), jnp.int32),
        pltpu.VMEM((REM,), jnp.float32),
        pltpu.VMEM((REM, D_HID), jnp.float32),
        pltpu.VMEM((ZD,), jnp.float32),
        pltpu.SemaphoreType.DMA((NB,)),
        pltpu.SemaphoreType.DMA((NB,)),
        pltpu.SemaphoreType.DMA((NB,)),
    ],
)


def kernel(x, edge_index, W1, a1_src, a1_dst, W2, a2_src, a2_dst):
    src = edge_index[0]
    dst = edge_index[1]
    h1, as1, ad1 = _tc_embed(x, W1, a1_src, a1_dst)
    p1, d1f = _sc_edge(h1, src, dst, as1, ad1)
    d1 = d1f.reshape(NC, N_PAD)[:, :N]
    h2, as2, ad2 = _tc_combine_embed(p1, d1, W2, a2_src, a2_dst)
    p2, d2f = _sc_edge(h2, src, dst, as2, ad2)
    d2 = d2f.reshape(NC, N_PAD)[:, :N]
    return _tc_finalize(p2, d2)


# speedup vs baseline: 1.1636x; 1.1636x over previous
import functools

import jax
import jax.numpy as jnp
from jax import lax
from jax.experimental import pallas as pl
from jax.experimental.pallas import tpu as pltpu
from jax.experimental.pallas import tpu_sc as plsc

N = 10000
E = 320000
D_IN = 128
D_HID = 64
D_OUT = 64
NEG_SLOPE = 0.2

NC = 2
NS = 16
NW = NC * NS
L = 16

CH = 128
EPT = E // NW
NFULL = EPT // CH
REM = EPT - NFULL * CH
NB = 3
ZD = 208
ROWS_PT = 624
ROWS_TAIL = N - ROWS_PT * NS
N_PAD = 10240


def _tc_embed_body(x_ref, w_ref, asv_ref, adv_ref, h_ref, s_ref, d_ref):
    h = jnp.dot(x_ref[...], w_ref[...], preferred_element_type=jnp.float32)
    h_ref[...] = h
    s_ref[...] = jnp.dot(h, asv_ref[...], preferred_element_type=jnp.float32)
    d_ref[...] = jnp.dot(h, adv_ref[...], preferred_element_type=jnp.float32)


def _tc_embed(x, W, a_src, a_dst):
    n = x.shape[0]
    dh = W.shape[1]
    h, s, d = pl.pallas_call(
        _tc_embed_body,
        out_shape=[
            jax.ShapeDtypeStruct((n, dh), jnp.float32),
            jax.ShapeDtypeStruct((n, 1), jnp.float32),
            jax.ShapeDtypeStruct((n, 1), jnp.float32),
        ],
    )(x, W, a_src[:, None], a_dst[:, None])
    return h, s[:, 0], d[:, 0]


def _tc_combine_embed_body(p_ref, d_ref, w_ref, asv_ref, adv_ref,
                           h_ref, s_ref, dd_ref):
    denom = d_ref[0] + d_ref[1] + 1e-16
    z = (p_ref[0] + p_ref[1]) / denom
    g = jnp.where(z > 0, z, jnp.exp(z) - 1.0)
    h = jnp.dot(g, w_ref[...], preferred_element_type=jnp.float32)
    h_ref[...] = h
    s_ref[...] = jnp.dot(h, asv_ref[...], preferred_element_type=jnp.float32)
    dd_ref[...] = jnp.dot(h, adv_ref[...], preferred_element_type=jnp.float32)


def _tc_combine_embed(p, d, W, a_src, a_dst):
    n = p.shape[1]
    dh = W.shape[1]
    h, s, dd = pl.pallas_call(
        _tc_combine_embed_body,
        out_shape=[
            jax.ShapeDtypeStruct((n, dh), jnp.float32),
            jax.ShapeDtypeStruct((n, 1), jnp.float32),
            jax.ShapeDtypeStruct((n, 1), jnp.float32),
        ],
    )(p, d[:, :, None], W, a_src[:, None], a_dst[:, None])
    return h, s[:, 0], dd[:, 0]


def _tc_finalize_body(p_ref, d_ref, o_ref):
    denom = d_ref[0] + d_ref[1] + 1e-16
    o_ref[...] = (p_ref[0] + p_ref[1]) / denom


def _tc_finalize(p, d):
    n = p.shape[1]
    dh = p.shape[2]
    return pl.pallas_call(
        _tc_finalize_body,
        out_shape=jax.ShapeDtypeStruct((n, dh), jnp.float32),
    )(p, d[:, :, None])


def _sc_edge_body(h_hbm, src_hbm, dst_hbm, as_hbm, ad_hbm,
                  p_hbm, dn_hbm,
                  h_sp, acc_sp, den_sp,
                  as_v, ad_v, srcq, dstq, w_q, rowsq, rows_v,
                  rsrc, rdst, rw, rrows, zd_v,
                  isem, gsem, ssem):
    cid = lax.axis_index("c")
    sid = lax.axis_index("s")
    wid = sid * NC + cid

    r0 = sid * ROWS_PT

    schunks = ((0, CH), (CH, CH), (2 * CH, CH), (3 * CH, CH), (4 * CH, 112))

    pltpu.async_copy(as_hbm, as_v, isem.at[2])
    pltpu.async_copy(ad_hbm, ad_v, isem.at[2])

    def _srd(i):
        off, sz = schunks[i]
        b = i % NB
        pltpu.async_copy(h_hbm.at[pl.ds(r0 + off, sz)],
                         rowsq.at[b, pl.ds(0, sz)], gsem.at[b])

    def _swt(i):
        off, sz = schunks[i]
        b = i % NB
        pltpu.make_async_copy(h_hbm.at[pl.ds(r0 + off, sz)],
                              rowsq.at[b, pl.ds(0, sz)], gsem.at[b]).wait()

    def _swr(i):
        off, sz = schunks[i]
        b = i % NB
        pltpu.async_copy(rowsq.at[b, pl.ds(0, sz)],
                         h_sp.at[pl.ds(r0 + off, sz)], ssem.at[b])

    def _sdw(i):
        off, sz = schunks[i]
        b = i % NB
        pltpu.make_async_copy(rowsq.at[b, pl.ds(0, sz)],
                              h_sp.at[pl.ds(r0 + off, sz)], ssem.at[b]).wait()

    for i in range(3):
        _srd(i)
    for i in range(3):
        _swt(i)
        _swr(i)
    for i in range(3, 5):
        _sdw(i - 3)
        _srd(i)
    for i in range(3, 5):
        _swt(i)
        _swr(i)
    for i in range(2, 5):
        _sdw(i)

    @pl.when(sid == NS - 1)
    def _tail_stage():
        t0 = N - ROWS_TAIL
        pltpu.sync_copy(h_hbm.at[pl.ds(t0, ROWS_TAIL)], rows_v)
        pltpu.sync_copy(rows_v, h_sp.at[pl.ds(t0, ROWS_TAIL)])

    zeros16 = jnp.zeros((L,), jnp.float32)

    def _zrow(i, _):
        for j in range(D_HID // L):
            rowsq[0, i, pl.ds(j * L, L)] = zeros16
        return _
    lax.fori_loop(0, CH, _zrow, None)

    def _zd(i, _):
        zd_v[pl.ds(i * L, L)] = zeros16
        return _
    lax.fori_loop(0, ZD // L, _zd, None)

    for i in range(5):
        off, sz = schunks[i]
        pltpu.async_copy(rowsq.at[0, pl.ds(0, sz)],
                         acc_sp.at[pl.ds(r0 + off, sz)], isem.at[0])
    for k in range(ROWS_PT // ZD):
        pltpu.async_copy(zd_v, den_sp.at[pl.ds(r0 + k * ZD, ZD)],
                         isem.at[1])

    @pl.when(sid == NS - 1)
    def _tail_zero():
        t0 = N - ROWS_TAIL
        pltpu.sync_copy(rowsq.at[0, pl.ds(0, ROWS_TAIL)],
                        acc_sp.at[pl.ds(t0, ROWS_TAIL)])
        pltpu.sync_copy(zd_v.at[pl.ds(0, ROWS_TAIL)],
                        den_sp.at[pl.ds(t0, ROWS_TAIL)])

    for i in range(5):
        off, sz = schunks[i]
        pltpu.make_async_copy(rowsq.at[0, pl.ds(0, sz)],
                              acc_sp.at[pl.ds(r0 + off, sz)],
                              isem.at[0]).wait()
    for k in range(ROWS_PT // ZD):
        pltpu.make_async_copy(zd_v, den_sp.at[pl.ds(r0 + k * ZD, ZD)],
                              isem.at[1]).wait()
    pltpu.make_async_copy(as_hbm, as_v, isem.at[2]).wait()
    pltpu.make_async_copy(ad_hbm, ad_v, isem.at[2]).wait()

    plsc.subcore_barrier()

    e0 = wid * EPT

    def _issue_idx(c, b):
        base = e0 + c * CH
        pltpu.async_copy(src_hbm.at[pl.ds(base, CH)], srcq.at[b], isem.at[b])
        pltpu.async_copy(dst_hbm.at[pl.ds(base, CH)], dstq.at[b], isem.at[b])

    def _wait_idx(c, b):
        base = e0 + c * CH
        pltpu.make_async_copy(
            src_hbm.at[pl.ds(base, CH)], srcq.at[b], isem.at[b]).wait()
        pltpu.make_async_copy(
            dst_hbm.at[pl.ds(base, CH)], dstq.at[b], isem.at[b]).wait()

    def _issue_gather(b):
        pltpu.async_copy(h_sp.at[srcq.at[b]], rowsq.at[b], gsem.at[b])

    def _wait_gather(b):
        pltpu.make_async_copy(
            h_sp.at[srcq.at[b]], rowsq.at[b], gsem.at[b]).wait()

    def _issue_scatter(b):
        pltpu.async_copy(rowsq.at[b], acc_sp.at[dstq.at[b]], ssem.at[b],
                         add=True)
        pltpu.async_copy(w_q.at[b], den_sp.at[dstq.at[b]], ssem.at[b],
                         add=True)

    def _drain_scatter(b):
        pltpu.make_async_copy(
            rowsq.at[b], acc_sp.at[dstq.at[b]], ssem.at[b]).wait()
        pltpu.make_async_copy(
            w_q.at[b], den_sp.at[dstq.at[b]], ssem.at[b]).wait()

    def _compute_w(b):
        @plsc.parallel_loop(0, CH // L, unroll=2)
        def _w(i):
            s16 = srcq[b, pl.ds(i * L, L)]
            d16 = dstq[b, pl.ds(i * L, L)]
            e = plsc.load_gather(as_v, [s16]) + plsc.load_gather(ad_v, [d16])
            e = jnp.where(e > 0, e, NEG_SLOPE * e)
            w_q[b, pl.ds(i * L, L)] = jnp.exp(e)

    def _scale_rows(b):
        @plsc.parallel_loop(0, CH, unroll=4)
        def _s(i):
            wv = plsc.load_gather(w_q.at[b], [jnp.full((L,), i, jnp.int32)])
            for j in range(D_HID // L):
                sl = pl.ds(j * L, L)
                rowsq[b, i, sl] = rowsq[b, i, sl] * wv

    _issue_idx(0, 0)
    _issue_idx(1, 1)
    _wait_idx(0, 0)
    _issue_gather(0)

    def _chunk(ci, _):
        b = lax.rem(ci, NB)
        s1 = lax.rem(ci + 1, NB)
        s2 = lax.rem(ci + 2, NB)
        _compute_w(b)
        _wait_gather(b)
        _scale_rows(b)
        _issue_scatter(b)

        @pl.when(ci + 1 < NFULL)
        def _prep_gather():
            _wait_idx(ci + 1, s1)
            _issue_gather(s1)

        @pl.when(ci >= NB - 2)
        def _drain_prev():
            _drain_scatter(s2)

        @pl.when(ci + 2 < NFULL)
        def _prep_idx():
            _issue_idx(ci + 2, s2)
        return _

    lax.fori_loop(0, NFULL, _chunk, None)

    for cd in range(NFULL - (NB - 2), NFULL):
        _drain_scatter(jnp.int32(cd % NB))

    rbase = e0 + NFULL * CH
    pltpu.sync_copy(src_hbm.at[pl.ds(rbase, REM)], rsrc)
    pltpu.sync_copy(dst_hbm.at[pl.ds(rbase, REM)], rdst)
    s16 = rsrc[...]
    d16 = rdst[...]
    e = plsc.load_gather(as_v, [s16]) + plsc.load_gather(ad_v, [d16])
    e = jnp.where(e > 0, e, NEG_SLOPE * e)
    rw[...] = jnp.exp(e)
    pltpu.sync_copy(h_sp.at[rsrc], rrows)

    @plsc.parallel_loop(0, REM, unroll=4)
    def _rscale(i):
        wv = plsc.load_gather(rw, [jnp.full((L,), i, jnp.int32)])
        for j in range(D_HID // L):
            sl = pl.ds(j * L, L)
            rrows[i, sl] = rrows[i, sl] * wv

    pltpu.sync_copy(rrows, acc_sp.at[rdst], add=True)
    pltpu.sync_copy(rw, den_sp.at[rdst], add=True)

    plsc.subcore_barrier()

    def _ord(i):
        off, sz = schunks[i]
        b = i % NB
        pltpu.async_copy(acc_sp.at[pl.ds(r0 + off, sz)],
                         rowsq.at[b, pl.ds(0, sz)], gsem.at[b])

    def _owt(i):
        off, sz = schunks[i]
        b = i % NB
        pltpu.make_async_copy(acc_sp.at[pl.ds(r0 + off, sz)],
                              rowsq.at[b, pl.ds(0, sz)], gsem.at[b]).wait()

    def _owr(i):
        off, sz = schunks[i]
        b = i % NB
        pltpu.async_copy(rowsq.at[b, pl.ds(0, sz)],
                         p_hbm.at[cid, pl.ds(r0 + off, sz)], ssem.at[b])

    def _odw(i):
        off, sz = schunks[i]
        b = i % NB
        pltpu.make_async_copy(rowsq.at[b, pl.ds(0, sz)],
                              p_hbm.at[cid, pl.ds(r0 + off, sz)],
                              ssem.at[b]).wait()

    for i in range(3):
        _ord(i)
    for i in range(3):
        _owt(i)
        _owr(i)
    for i in range(3, 5):
        _odw(i - 3)
        _ord(i)
    for i in range(3, 5):
        _owt(i)
        _owr(i)

    def _wbden(i, _):
        pltpu.sync_copy(den_sp.at[pl.ds(r0 + i * ZD, ZD)], zd_v)
        pltpu.sync_copy(zd_v, dn_hbm.at[pl.ds(cid * N_PAD + r0 + i * ZD, ZD)])
        return _
    lax.fori_loop(0, ROWS_PT // ZD, _wbden, None)

    @pl.when(sid == NS - 1)
    def _tail_out():
        t0 = N - ROWS_TAIL
        pltpu.sync_copy(acc_sp.at[pl.ds(t0, ROWS_TAIL)], rows_v)
        pltpu.sync_copy(rows_v, p_hbm.at[cid, pl.ds(t0, ROWS_TAIL)])
        pltpu.sync_copy(den_sp.at[pl.ds(t0, ROWS_TAIL)],
                        zd_v.at[pl.ds(0, ROWS_TAIL)])
        pltpu.sync_copy(zd_v.at[pl.ds(0, ROWS_TAIL)],
                        dn_hbm.at[pl.ds(cid * N_PAD + t0, ROWS_TAIL)])

    for i in range(2, 5):
        _odw(i)


_sc_edge = pl.kernel(
    _sc_edge_body,
    out_type=(
        jax.ShapeDtypeStruct((NC, N, D_HID), jnp.float32),
        jax.ShapeDtypeStruct((NC * N_PAD,), jnp.float32),
    ),
    mesh=plsc.VectorSubcoreMesh(
        core_axis_name="c", subcore_axis_name="s",
        num_cores=NC, num_subcores=NS),
    compiler_params=pltpu.CompilerParams(
        needs_layout_passes=False, use_tc_tiling_on_sc=False),
    scratch_types=[
        pltpu.VMEM_SHARED((N, D_HID), jnp.float32),
        pltpu.VMEM_SHARED((N, D_HID), jnp.float32),
        pltpu.VMEM_SHARED((N,), jnp.float32),
        pltpu.VMEM((N,), jnp.float32),
        pltpu.VMEM((N,), jnp.float32),
        pltpu.VMEM((NB, CH), jnp.int32),
        pltpu.VMEM((NB, CH), jnp.int32),
        pltpu.VMEM((NB, CH), jnp.float32),
        pltpu.VMEM((NB, CH, D_HID), jnp.float32),
        pltpu.VMEM((ROWS_TAIL, D_HID), jnp.float32),
        pltpu.VMEM((REM,), jnp.int32),
        pltpu.VMEM((REM,), jnp.int32),
        pltpu.VMEM((REM,), jnp.float32),
        pltpu.VMEM((REM, D_HID), jnp.float32),
        pltpu.VMEM((ZD,), jnp.float32),
        pltpu.SemaphoreType.DMA((NB,)),
        pltpu.SemaphoreType.DMA((NB,)),
        pltpu.SemaphoreType.DMA((NB,)),
    ],
)


def kernel(x, edge_index, W1, a1_src, a1_dst, W2, a2_src, a2_dst):
    src = edge_index[0]
    dst = edge_index[1]
    h1, as1, ad1 = _tc_embed(x, W1, a1_src, a1_dst)
    p1, d1f = _sc_edge(h1, src, dst, as1, ad1)
    d1 = d1f.reshape(NC, N_PAD)[:, :N]
    h2, as2, ad2 = _tc_combine_embed(p1, d1, W2, a2_src, a2_dst)
    p2, d2f = _sc_edge(h2, src, dst, as2, ad2)
    d2 = d2f.reshape(NC, N_PAD)[:, :N]
    return _tc_finalize(p2, d2)


# speedup vs baseline: 1.2130x; 1.0425x over previous
import jax
import jax.numpy as jnp
from jax import lax
from jax.experimental import pallas as pl
from jax.experimental.pallas import tpu as pltpu
from jax.experimental.pallas import tpu_sc as plsc

N = 10000
E = 320000
D_IN = 128
D_HID = 64
D_OUT = 64
NEG_SLOPE = 0.2

NC = 2
NS = 16
NW = NC * NS
L = 16

CH = 128
EPT = E // NW
NFULL = EPT // CH
REM = EPT - NFULL * CH
NB = 3
ZD = 208
ROWS_PT = 624
ROWS_TAIL = N - ROWS_PT * NS
N_PAD = 10240


def _tc_embed_body(x_ref, w_ref, asv_ref, adv_ref, h_ref, s_ref, d_ref):
    h = jnp.dot(x_ref[...], w_ref[...], preferred_element_type=jnp.float32)
    h_ref[...] = h
    s_ref[...] = jnp.sum(h * asv_ref[...][None, :], axis=1)
    d_ref[...] = jnp.sum(h * adv_ref[...][None, :], axis=1)


def _tc_embed(x, W, a_src, a_dst):
    n = x.shape[0]
    dh = W.shape[1]
    return pl.pallas_call(
        _tc_embed_body,
        out_shape=[
            jax.ShapeDtypeStruct((n, dh), jnp.float32),
            jax.ShapeDtypeStruct((n,), jnp.float32),
            jax.ShapeDtypeStruct((n,), jnp.float32),
        ],
    )(x, W, a_src, a_dst)


def _tc_combine_embed_body(p_ref, d_ref, w_ref, asv_ref, adv_ref,
                           h_ref, s_ref, dd_ref):
    denom = d_ref[0] + d_ref[1] + 1e-16
    z = (p_ref[0] + p_ref[1]) / denom
    g = jnp.where(z > 0, z, jnp.exp(z) - 1.0)
    h = jnp.dot(g, w_ref[...], preferred_element_type=jnp.float32)
    h_ref[...] = h
    s_ref[...] = jnp.sum(h * asv_ref[...][None, :], axis=1)
    d_ref2 = adv_ref[...][None, :]
    dd_ref[...] = jnp.sum(h * d_ref2, axis=1)


def _tc_combine_embed(p, d, W, a_src, a_dst):
    n = p.shape[1]
    dh = W.shape[1]
    return pl.pallas_call(
        _tc_combine_embed_body,
        out_shape=[
            jax.ShapeDtypeStruct((n, dh), jnp.float32),
            jax.ShapeDtypeStruct((n,), jnp.float32),
            jax.ShapeDtypeStruct((n,), jnp.float32),
        ],
    )(p, d[:, :, None], W, a_src, a_dst)


def _tc_finalize_body(p_ref, d_ref, o_ref):
    denom = d_ref[0] + d_ref[1] + 1e-16
    o_ref[...] = (p_ref[0] + p_ref[1]) / denom


def _tc_finalize(p, d):
    n = p.shape[1]
    dh = p.shape[2]
    return pl.pallas_call(
        _tc_finalize_body,
        out_shape=jax.ShapeDtypeStruct((n, dh), jnp.float32),
    )(p, d[:, :, None])


def _sc_edge_body(h_hbm, src_hbm, dst_hbm, as_hbm, ad_hbm,
                  p_hbm, dn_hbm,
                  h_sp, acc_sp, den_sp,
                  as_v, ad_v, srcq, dstq, w_q, rowsq, rows_v,
                  rsrc, rdst, rw, rrows, zd_v,
                  isem, gsem, ssem):
    cid = lax.axis_index("c")
    sid = lax.axis_index("s")
    wid = sid * NC + cid

    r0 = sid * ROWS_PT

    schunks = ((0, CH), (CH, CH), (2 * CH, CH), (3 * CH, CH), (4 * CH, 112))

    pltpu.async_copy(as_hbm, as_v, isem.at[2])
    pltpu.async_copy(ad_hbm, ad_v, isem.at[2])

    def _srd(i):
        off, sz = schunks[i]
        b = i % NB
        pltpu.async_copy(h_hbm.at[pl.ds(r0 + off, sz)],
                         rowsq.at[b, pl.ds(0, sz)], gsem.at[b])

    def _swt(i):
        off, sz = schunks[i]
        b = i % NB
        pltpu.make_async_copy(h_hbm.at[pl.ds(r0 + off, sz)],
                              rowsq.at[b, pl.ds(0, sz)], gsem.at[b]).wait()

    def _swr(i):
        off, sz = schunks[i]
        b = i % NB
        pltpu.async_copy(rowsq.at[b, pl.ds(0, sz)],
                         h_sp.at[pl.ds(r0 + off, sz)], ssem.at[b])

    def _sdw(i):
        off, sz = schunks[i]
        b = i % NB
        pltpu.make_async_copy(rowsq.at[b, pl.ds(0, sz)],
                              h_sp.at[pl.ds(r0 + off, sz)], ssem.at[b]).wait()

    for i in range(3):
        _srd(i)
    for i in range(3):
        _swt(i)
        _swr(i)
    for i in range(3, 5):
        _sdw(i - 3)
        _srd(i)
    for i in range(3, 5):
        _swt(i)
        _swr(i)
    for i in range(2, 5):
        _sdw(i)

    @pl.when(sid == NS - 1)
    def _tail_stage():
        t0 = N - ROWS_TAIL
        pltpu.sync_copy(h_hbm.at[pl.ds(t0, ROWS_TAIL)], rows_v)
        pltpu.sync_copy(rows_v, h_sp.at[pl.ds(t0, ROWS_TAIL)])

    zeros16 = jnp.zeros((L,), jnp.float32)

    def _zrow(i, _):
        for j in range(D_HID // L):
            rowsq[0, i, pl.ds(j * L, L)] = zeros16
        return _
    lax.fori_loop(0, CH, _zrow, None)

    def _zd(i, _):
        zd_v[pl.ds(i * L, L)] = zeros16
        return _
    lax.fori_loop(0, ZD // L, _zd, None)

    for i in range(5):
        off, sz = schunks[i]
        pltpu.async_copy(rowsq.at[0, pl.ds(0, sz)],
                         acc_sp.at[pl.ds(r0 + off, sz)], isem.at[0])
    for k in range(ROWS_PT // ZD):
        pltpu.async_copy(zd_v, den_sp.at[pl.ds(r0 + k * ZD, ZD)],
                         isem.at[1])

    @pl.when(sid == NS - 1)
    def _tail_zero():
        t0 = N - ROWS_TAIL
        pltpu.sync_copy(rowsq.at[0, pl.ds(0, ROWS_TAIL)],
                        acc_sp.at[pl.ds(t0, ROWS_TAIL)])
        pltpu.sync_copy(zd_v.at[pl.ds(0, ROWS_TAIL)],
                        den_sp.at[pl.ds(t0, ROWS_TAIL)])

    for i in range(5):
        off, sz = schunks[i]
        pltpu.make_async_copy(rowsq.at[0, pl.ds(0, sz)],
                              acc_sp.at[pl.ds(r0 + off, sz)],
                              isem.at[0]).wait()
    for k in range(ROWS_PT // ZD):
        pltpu.make_async_copy(zd_v, den_sp.at[pl.ds(r0 + k * ZD, ZD)],
                              isem.at[1]).wait()
    pltpu.make_async_copy(as_hbm, as_v, isem.at[2]).wait()
    pltpu.make_async_copy(ad_hbm, ad_v, isem.at[2]).wait()

    plsc.subcore_barrier()

    e0 = wid * EPT

    def _issue_idx(c, b):
        base = e0 + c * CH
        pltpu.async_copy(src_hbm.at[pl.ds(base, CH)], srcq.at[b], isem.at[b])
        pltpu.async_copy(dst_hbm.at[pl.ds(base, CH)], dstq.at[b], isem.at[b])

    def _wait_idx(c, b):
        base = e0 + c * CH
        pltpu.make_async_copy(
            src_hbm.at[pl.ds(base, CH)], srcq.at[b], isem.at[b]).wait()
        pltpu.make_async_copy(
            dst_hbm.at[pl.ds(base, CH)], dstq.at[b], isem.at[b]).wait()

    def _issue_gather(b):
        pltpu.async_copy(h_sp.at[srcq.at[b]], rowsq.at[b], gsem.at[b])

    def _wait_gather(b):
        pltpu.make_async_copy(
            h_sp.at[srcq.at[b]], rowsq.at[b], gsem.at[b]).wait()

    def _issue_scatter(b):
        pltpu.async_copy(rowsq.at[b], acc_sp.at[dstq.at[b]], ssem.at[b],
                         add=True)
        pltpu.async_copy(w_q.at[b], den_sp.at[dstq.at[b]], ssem.at[b],
                         add=True)

    def _drain_scatter(b):
        pltpu.make_async_copy(
            rowsq.at[b], acc_sp.at[dstq.at[b]], ssem.at[b]).wait()
        pltpu.make_async_copy(
            w_q.at[b], den_sp.at[dstq.at[b]], ssem.at[b]).wait()

    def _compute_w(b):
        @plsc.parallel_loop(0, CH // L, unroll=2)
        def _w(i):
            s16 = srcq[b, pl.ds(i * L, L)]
            d16 = dstq[b, pl.ds(i * L, L)]
            e = plsc.load_gather(as_v, [s16]) + plsc.load_gather(ad_v, [d16])
            e = jnp.where(e > 0, e, NEG_SLOPE * e)
            w_q[b, pl.ds(i * L, L)] = jnp.exp(e)

    def _scale_rows(b):
        @plsc.parallel_loop(0, CH, unroll=4)
        def _s(i):
            wv = plsc.load_gather(w_q.at[b], [jnp.full((L,), i, jnp.int32)])
            for j in range(D_HID // L):
                sl = pl.ds(j * L, L)
                rowsq[b, i, sl] = rowsq[b, i, sl] * wv

    _issue_idx(0, 0)
    _issue_idx(1, 1)
    _wait_idx(0, 0)
    _issue_gather(0)

    def _chunk(ci, _):
        b = lax.rem(ci, NB)
        s1 = lax.rem(ci + 1, NB)
        s2 = lax.rem(ci + 2, NB)
        _compute_w(b)
        _wait_gather(b)
        _scale_rows(b)
        _issue_scatter(b)

        @pl.when(ci + 1 < NFULL)
        def _prep_gather():
            _wait_idx(ci + 1, s1)
            _issue_gather(s1)

        @pl.when(ci >= NB - 2)
        def _drain_prev():
            _drain_scatter(s2)

        @pl.when(ci + 2 < NFULL)
        def _prep_idx():
            _issue_idx(ci + 2, s2)
        return _

    lax.fori_loop(0, NFULL, _chunk, None)

    for cd in range(NFULL - (NB - 2), NFULL):
        _drain_scatter(jnp.int32(cd % NB))

    rbase = e0 + NFULL * CH
    pltpu.sync_copy(src_hbm.at[pl.ds(rbase, REM)], rsrc)
    pltpu.sync_copy(dst_hbm.at[pl.ds(rbase, REM)], rdst)
    s16 = rsrc[...]
    d16 = rdst[...]
    e = plsc.load_gather(as_v, [s16]) + plsc.load_gather(ad_v, [d16])
    e = jnp.where(e > 0, e, NEG_SLOPE * e)
    rw[...] = jnp.exp(e)
    pltpu.sync_copy(h_sp.at[rsrc], rrows)

    @plsc.parallel_loop(0, REM, unroll=4)
    def _rscale(i):
        wv = plsc.load_gather(rw, [jnp.full((L,), i, jnp.int32)])
        for j in range(D_HID // L):
            sl = pl.ds(j * L, L)
            rrows[i, sl] = rrows[i, sl] * wv

    pltpu.sync_copy(rrows, acc_sp.at[rdst], add=True)
    pltpu.sync_copy(rw, den_sp.at[rdst], add=True)

    plsc.subcore_barrier()

    def _ord(i):
        off, sz = schunks[i]
        b = i % NB
        pltpu.async_copy(acc_sp.at[pl.ds(r0 + off, sz)],
                         rowsq.at[b, pl.ds(0, sz)], gsem.at[b])

    def _owt(i):
        off, sz = schunks[i]
        b = i % NB
        pltpu.make_async_copy(acc_sp.at[pl.ds(r0 + off, sz)],
                              rowsq.at[b, pl.ds(0, sz)], gsem.at[b]).wait()

    def _owr(i):
        off, sz = schunks[i]
        b = i % NB
        pltpu.async_copy(rowsq.at[b, pl.ds(0, sz)],
                         p_hbm.at[cid, pl.ds(r0 + off, sz)], ssem.at[b])

    def _odw(i):
        off, sz = schunks[i]
        b = i % NB
        pltpu.make_async_copy(rowsq.at[b, pl.ds(0, sz)],
                              p_hbm.at[cid, pl.ds(r0 + off, sz)],
                              ssem.at[b]).wait()

    for i in range(3):
        _ord(i)
    for i in range(3):
        _owt(i)
        _owr(i)
    for i in range(3, 5):
        _odw(i - 3)
        _ord(i)
    for i in range(3, 5):
        _owt(i)
        _owr(i)

    def _wbden(i, _):
        pltpu.sync_copy(den_sp.at[pl.ds(r0 + i * ZD, ZD)], zd_v)
        pltpu.sync_copy(zd_v, dn_hbm.at[pl.ds(cid * N_PAD + r0 + i * ZD, ZD)])
        return _
    lax.fori_loop(0, ROWS_PT // ZD, _wbden, None)

    @pl.when(sid == NS - 1)
    def _tail_out():
        t0 = N - ROWS_TAIL
        pltpu.sync_copy(acc_sp.at[pl.ds(t0, ROWS_TAIL)], rows_v)
        pltpu.sync_copy(rows_v, p_hbm.at[cid, pl.ds(t0, ROWS_TAIL)])
        pltpu.sync_copy(den_sp.at[pl.ds(t0, ROWS_TAIL)],
                        zd_v.at[pl.ds(0, ROWS_TAIL)])
        pltpu.sync_copy(zd_v.at[pl.ds(0, ROWS_TAIL)],
                        dn_hbm.at[pl.ds(cid * N_PAD + t0, ROWS_TAIL)])

    for i in range(2, 5):
        _odw(i)


_sc_edge = pl.kernel(
    _sc_edge_body,
    out_type=(
        jax.ShapeDtypeStruct((NC, N, D_HID), jnp.float32),
        jax.ShapeDtypeStruct((NC * N_PAD,), jnp.float32),
    ),
    mesh=plsc.VectorSubcoreMesh(
        core_axis_name="c", subcore_axis_name="s",
        num_cores=NC, num_subcores=NS),
    compiler_params=pltpu.CompilerParams(
        needs_layout_passes=False, use_tc_tiling_on_sc=False),
    scratch_types=[
        pltpu.VMEM_SHARED((N, D_HID), jnp.float32),
        pltpu.VMEM_SHARED((N, D_HID), jnp.float32),
        pltpu.VMEM_SHARED((N,), jnp.float32),
        pltpu.VMEM((N,), jnp.float32),
        pltpu.VMEM((N,), jnp.float32),
        pltpu.VMEM((NB, CH), jnp.int32),
        pltpu.VMEM((NB, CH), jnp.int32),
        pltpu.VMEM((NB, CH), jnp.float32),
        pltpu.VMEM((NB, CH, D_HID), jnp.float32),
        pltpu.VMEM((ROWS_TAIL, D_HID), jnp.float32),
        pltpu.VMEM((REM,), jnp.int32),
        pltpu.VMEM((REM,), jnp.int32),
        pltpu.VMEM((REM,), jnp.float32),
        pltpu.VMEM((REM, D_HID), jnp.float32),
        pltpu.VMEM((ZD,), jnp.float32),
        pltpu.SemaphoreType.DMA((NB,)),
        pltpu.SemaphoreType.DMA((NB,)),
        pltpu.SemaphoreType.DMA((NB,)),
    ],
)


def kernel(x, edge_index, W1, a1_src, a1_dst, W2, a2_src, a2_dst):
    src = edge_index[0]
    dst = edge_index[1]
    h1, as1, ad1 = _tc_embed(x, W1, a1_src, a1_dst)
    p1, d1f = _sc_edge(h1, src, dst, as1, ad1)
    d1 = d1f.reshape(NC, N_PAD)[:, :N]
    h2, as2, ad2 = _tc_combine_embed(p1, d1, W2, a2_src, a2_dst)
    p2, d2f = _sc_edge(h2, src, dst, as2, ad2)
    d2 = d2f.reshape(NC, N_PAD)[:, :N]
    return _tc_finalize(p2, d2)


# speedup vs baseline: 1.2462x; 1.0274x over previous
import jax
import jax.numpy as jnp
from jax import lax
from jax.experimental import pallas as pl
from jax.experimental.pallas import tpu as pltpu
from jax.experimental.pallas import tpu_sc as plsc

N = 10000
E = 320000
D_IN = 128
D_HID = 64
D_OUT = 64
NEG_SLOPE = 0.2

NC = 2
NS = 16
NW = NC * NS
L = 16

CH = 128
EPT = E // NW
NFULL = EPT // CH
REM = EPT - NFULL * CH
NB = 3
ZD = 208
ROWS_PT = 624
ROWS_TAIL = N - ROWS_PT * NS
N_PAD = 10240


def _tc_embed_body(x_ref, w_ref, asv_ref, adv_ref, h_ref, s_ref, d_ref):
    h = jnp.dot(x_ref[...], w_ref[...], preferred_element_type=jnp.float32)
    h_ref[...] = h
    s_ref[...] = jnp.sum(h * asv_ref[...][None, :], axis=1)
    d_ref[...] = jnp.sum(h * adv_ref[...][None, :], axis=1)


def _tc_embed(x, W, a_src, a_dst):
    n = x.shape[0]
    dh = W.shape[1]
    return pl.pallas_call(
        _tc_embed_body,
        out_shape=[
            jax.ShapeDtypeStruct((n, dh), jnp.float32),
            jax.ShapeDtypeStruct((n,), jnp.float32),
            jax.ShapeDtypeStruct((n,), jnp.float32),
        ],
    )(x, W, a_src, a_dst)


def _tc_elu_embed_body(z_ref, w_ref, asv_ref, adv_ref, h_ref, s_ref, d_ref):
    z = z_ref[...]
    g = jnp.where(z > 0, z, jnp.exp(z) - 1.0)
    h = jnp.dot(g, w_ref[...], preferred_element_type=jnp.float32)
    h_ref[...] = h
    s_ref[...] = jnp.sum(h * asv_ref[...][None, :], axis=1)
    d_ref[...] = jnp.sum(h * adv_ref[...][None, :], axis=1)


def _tc_elu_embed(z, W, a_src, a_dst):
    n = z.shape[0]
    dh = W.shape[1]
    return pl.pallas_call(
        _tc_elu_embed_body,
        out_shape=[
            jax.ShapeDtypeStruct((n, dh), jnp.float32),
            jax.ShapeDtypeStruct((n,), jnp.float32),
            jax.ShapeDtypeStruct((n,), jnp.float32),
        ],
    )(z, W, a_src, a_dst)


_CCHUNKS = ((0, 128), (128, 128), (256, 56))


def _sc_comb_body(p_hbm, dn_hbm, z_hbm,
                  paq, pbq, daq, dbq, rcq, rsem, wsem):
    sid = lax.axis_index("s")
    cid = lax.axis_index("c")
    r0 = sid * ROWS_PT + cid * (ROWS_PT // 2)

    def _rd(i):
        off, sz = _CCHUNKS[i]
        b = i % 3
        pltpu.sync_copy(p_hbm.at[0, pl.ds(r0 + off, sz)],
                        paq.at[b, pl.ds(0, sz)])
        pltpu.sync_copy(p_hbm.at[1, pl.ds(r0 + off, sz)],
                        pbq.at[b, pl.ds(0, sz)])
        pltpu.sync_copy(dn_hbm.at[pl.ds(r0 + off, sz)],
                        daq.at[b, pl.ds(0, sz)])
        pltpu.sync_copy(dn_hbm.at[pl.ds(N_PAD + r0 + off, sz)],
                        dbq.at[b, pl.ds(0, sz)])

    def _wr(i):
        off, sz = _CCHUNKS[i]
        b = i % 3
        pltpu.sync_copy(paq.at[b, pl.ds(0, sz)],
                        z_hbm.at[pl.ds(r0 + off, sz)])

    def _compute(i):
        off, sz = _CCHUNKS[i]
        b = i % 3

        @plsc.parallel_loop(0, (sz + L - 1) // L, unroll=2)
        def _rc(k):
            sl = pl.ds(k * L, L)
            rcq[b, sl] = 1.0 / (daq[b, sl] + dbq[b, sl] + 1e-16)

        @plsc.parallel_loop(0, sz, unroll=4)
        def _row(i2):
            rv = plsc.load_gather(rcq.at[b], [jnp.full((L,), i2, jnp.int32)])
            for j in range(D_HID // L):
                sl = pl.ds(j * L, L)
                paq[b, i2, sl] = (paq[b, i2, sl] + pbq[b, i2, sl]) * rv

    for i in range(3):
        _rd(i)
        _compute(i)
        _wr(i)

    @pl.when(jnp.logical_and(cid == 0, sid == NS - 1))
    def _tail():
        t0 = N - ROWS_TAIL
        pltpu.sync_copy(p_hbm.at[0, pl.ds(t0, ROWS_TAIL)],
                        paq.at[0, pl.ds(0, ROWS_TAIL)])
        pltpu.sync_copy(p_hbm.at[1, pl.ds(t0, ROWS_TAIL)],
                        pbq.at[0, pl.ds(0, ROWS_TAIL)])
        pltpu.sync_copy(dn_hbm.at[pl.ds(t0, ROWS_TAIL)],
                        daq.at[0, pl.ds(0, ROWS_TAIL)])
        pltpu.sync_copy(dn_hbm.at[pl.ds(N_PAD + t0, ROWS_TAIL)],
                        dbq.at[0, pl.ds(0, ROWS_TAIL)])

        @plsc.parallel_loop(0, ROWS_TAIL // L, unroll=1)
        def _rc(k):
            sl = pl.ds(k * L, L)
            rcq[0, sl] = 1.0 / (daq[0, sl] + dbq[0, sl] + 1e-16)

        @plsc.parallel_loop(0, ROWS_TAIL, unroll=4)
        def _row(i2):
            rv = plsc.load_gather(rcq.at[0],
                                  [jnp.full((L,), i2, jnp.int32)])
            for j in range(D_HID // L):
                sl = pl.ds(j * L, L)
                paq[0, i2, sl] = (paq[0, i2, sl] + pbq[0, i2, sl]) * rv

        pltpu.sync_copy(paq.at[0, pl.ds(0, ROWS_TAIL)],
                        z_hbm.at[pl.ds(t0, ROWS_TAIL)])


_sc_comb = pl.kernel(
    _sc_comb_body,
    out_type=jax.ShapeDtypeStruct((N, D_HID), jnp.float32),
    mesh=plsc.VectorSubcoreMesh(
        core_axis_name="c", subcore_axis_name="s",
        num_cores=NC, num_subcores=NS),
    compiler_params=pltpu.CompilerParams(
        needs_layout_passes=False, use_tc_tiling_on_sc=False),
    scratch_types=[
        pltpu.VMEM((3, 128, D_HID), jnp.float32),
        pltpu.VMEM((3, 128, D_HID), jnp.float32),
        pltpu.VMEM((3, 128), jnp.float32),
        pltpu.VMEM((3, 128), jnp.float32),
        pltpu.VMEM((3, 128), jnp.float32),
        pltpu.SemaphoreType.DMA((3,)),
        pltpu.SemaphoreType.DMA((3,)),
    ],
)


def _sc_edge_body(h_hbm, src_hbm, dst_hbm, as_hbm, ad_hbm,
                  p_hbm, dn_hbm,
                  h_sp, acc_sp, den_sp,
                  as_v, ad_v, srcq, dstq, w_q, rowsq, rows_v,
                  rsrc, rdst, rw, rrows, zd_v,
                  isem, gsem, ssem):
    cid = lax.axis_index("c")
    sid = lax.axis_index("s")
    wid = sid * NC + cid

    r0 = sid * ROWS_PT

    schunks = ((0, CH), (CH, CH), (2 * CH, CH), (3 * CH, CH), (4 * CH, 112))

    pltpu.async_copy(as_hbm, as_v, isem.at[2])
    pltpu.async_copy(ad_hbm, ad_v, isem.at[2])

    def _srd(i):
        off, sz = schunks[i]
        b = i % NB
        pltpu.async_copy(h_hbm.at[pl.ds(r0 + off, sz)],
                         rowsq.at[b, pl.ds(0, sz)], gsem.at[b])

    def _swt(i):
        off, sz = schunks[i]
        b = i % NB
        pltpu.make_async_copy(h_hbm.at[pl.ds(r0 + off, sz)],
                              rowsq.at[b, pl.ds(0, sz)], gsem.at[b]).wait()

    def _swr(i):
        off, sz = schunks[i]
        b = i % NB
        pltpu.async_copy(rowsq.at[b, pl.ds(0, sz)],
                         h_sp.at[pl.ds(r0 + off, sz)], ssem.at[b])

    def _sdw(i):
        off, sz = schunks[i]
        b = i % NB
        pltpu.make_async_copy(rowsq.at[b, pl.ds(0, sz)],
                              h_sp.at[pl.ds(r0 + off, sz)], ssem.at[b]).wait()

    for i in range(3):
        _srd(i)
    for i in range(3):
        _swt(i)
        _swr(i)
    for i in range(3, 5):
        _sdw(i - 3)
        _srd(i)
    for i in range(3, 5):
        _swt(i)
        _swr(i)
    for i in range(2, 5):
        _sdw(i)

    @pl.when(sid == NS - 1)
    def _tail_stage():
        t0 = N - ROWS_TAIL
        pltpu.sync_copy(h_hbm.at[pl.ds(t0, ROWS_TAIL)], rows_v)
        pltpu.sync_copy(rows_v, h_sp.at[pl.ds(t0, ROWS_TAIL)])

    zeros16 = jnp.zeros((L,), jnp.float32)

    def _zrow(i, _):
        for j in range(D_HID // L):
            rowsq[0, i, pl.ds(j * L, L)] = zeros16
        return _
    lax.fori_loop(0, CH, _zrow, None)

    def _zd(i, _):
        zd_v[pl.ds(i * L, L)] = zeros16
        return _
    lax.fori_loop(0, ZD // L, _zd, None)

    for i in range(5):
        off, sz = schunks[i]
        pltpu.async_copy(rowsq.at[0, pl.ds(0, sz)],
                         acc_sp.at[pl.ds(r0 + off, sz)], isem.at[0])
    for k in range(ROWS_PT // ZD):
        pltpu.async_copy(zd_v, den_sp.at[pl.ds(r0 + k * ZD, ZD)],
                         isem.at[1])

    @pl.when(sid == NS - 1)
    def _tail_zero():
        t0 = N - ROWS_TAIL
        pltpu.sync_copy(rowsq.at[0, pl.ds(0, ROWS_TAIL)],
                        acc_sp.at[pl.ds(t0, ROWS_TAIL)])
        pltpu.sync_copy(zd_v.at[pl.ds(0, ROWS_TAIL)],
                        den_sp.at[pl.ds(t0, ROWS_TAIL)])

    for i in range(5):
        off, sz = schunks[i]
        pltpu.make_async_copy(rowsq.at[0, pl.ds(0, sz)],
                              acc_sp.at[pl.ds(r0 + off, sz)],
                              isem.at[0]).wait()
    for k in range(ROWS_PT // ZD):
        pltpu.make_async_copy(zd_v, den_sp.at[pl.ds(r0 + k * ZD, ZD)],
                              isem.at[1]).wait()
    pltpu.make_async_copy(as_hbm, as_v, isem.at[2]).wait()
    pltpu.make_async_copy(ad_hbm, ad_v, isem.at[2]).wait()

    plsc.subcore_barrier()

    e0 = wid * EPT

    def _issue_idx(c, b):
        base = e0 + c * CH
        pltpu.async_copy(src_hbm.at[pl.ds(base, CH)], srcq.at[b], isem.at[b])
        pltpu.async_copy(dst_hbm.at[pl.ds(base, CH)], dstq.at[b], isem.at[b])

    def _wait_idx(c, b):
        base = e0 + c * CH
        pltpu.make_async_copy(
            src_hbm.at[pl.ds(base, CH)], srcq.at[b], isem.at[b]).wait()
        pltpu.make_async_copy(
            dst_hbm.at[pl.ds(base, CH)], dstq.at[b], isem.at[b]).wait()

    def _issue_gather(b):
        pltpu.async_copy(h_sp.at[srcq.at[b]], rowsq.at[b], gsem.at[b])

    def _wait_gather(b):
        pltpu.make_async_copy(
            h_sp.at[srcq.at[b]], rowsq.at[b], gsem.at[b]).wait()

    def _issue_scatter(b):
        pltpu.async_copy(rowsq.at[b], acc_sp.at[dstq.at[b]], ssem.at[b],
                         add=True)
        pltpu.async_copy(w_q.at[b], den_sp.at[dstq.at[b]], ssem.at[b],
                         add=True)

    def _drain_scatter(b):
        pltpu.make_async_copy(
            rowsq.at[b], acc_sp.at[dstq.at[b]], ssem.at[b]).wait()
        pltpu.make_async_copy(
            w_q.at[b], den_sp.at[dstq.at[b]], ssem.at[b]).wait()

    def _compute_w(b):
        @plsc.parallel_loop(0, CH // L, unroll=2)
        def _w(i):
            s16 = srcq[b, pl.ds(i * L, L)]
            d16 = dstq[b, pl.ds(i * L, L)]
            e = plsc.load_gather(as_v, [s16]) + plsc.load_gather(ad_v, [d16])
            e = jnp.where(e > 0, e, NEG_SLOPE * e)
            w_q[b, pl.ds(i * L, L)] = jnp.exp(e)

    def _scale_rows(b):
        @plsc.parallel_loop(0, CH, unroll=4)
        def _s(i):
            wv = plsc.load_gather(w_q.at[b], [jnp.full((L,), i, jnp.int32)])
            for j in range(D_HID // L):
                sl = pl.ds(j * L, L)
                rowsq[b, i, sl] = rowsq[b, i, sl] * wv

    _issue_idx(0, 0)
    _issue_idx(1, 1)
    _wait_idx(0, 0)
    _issue_gather(0)

    def _chunk(ci, _):
        b = lax.rem(ci, NB)
        s1 = lax.rem(ci + 1, NB)
        s2 = lax.rem(ci + 2, NB)
        _compute_w(b)
        _wait_gather(b)
        _scale_rows(b)
        _issue_scatter(b)

        @pl.when(ci + 1 < NFULL)
        def _prep_gather():
            _wait_idx(ci + 1, s1)
            _issue_gather(s1)

        @pl.when(ci >= NB - 2)
        def _drain_prev():
            _drain_scatter(s2)

        @pl.when(ci + 2 < NFULL)
        def _prep_idx():
            _issue_idx(ci + 2, s2)
        return _

    lax.fori_loop(0, NFULL, _chunk, None)

    for cd in range(NFULL - (NB - 2), NFULL):
        _drain_scatter(jnp.int32(cd % NB))

    rbase = e0 + NFULL * CH
    pltpu.sync_copy(src_hbm.at[pl.ds(rbase, REM)], rsrc)
    pltpu.sync_copy(dst_hbm.at[pl.ds(rbase, REM)], rdst)
    s16 = rsrc[...]
    d16 = rdst[...]
    e = plsc.load_gather(as_v, [s16]) + plsc.load_gather(ad_v, [d16])
    e = jnp.where(e > 0, e, NEG_SLOPE * e)
    rw[...] = jnp.exp(e)
    pltpu.sync_copy(h_sp.at[rsrc], rrows)

    @plsc.parallel_loop(0, REM, unroll=4)
    def _rscale(i):
        wv = plsc.load_gather(rw, [jnp.full((L,), i, jnp.int32)])
        for j in range(D_HID // L):
            sl = pl.ds(j * L, L)
            rrows[i, sl] = rrows[i, sl] * wv

    pltpu.sync_copy(rrows, acc_sp.at[rdst], add=True)
    pltpu.sync_copy(rw, den_sp.at[rdst], add=True)

    plsc.subcore_barrier()

    def _ord(i):
        off, sz = schunks[i]
        b = i % NB
        pltpu.async_copy(acc_sp.at[pl.ds(r0 + off, sz)],
                         rowsq.at[b, pl.ds(0, sz)], gsem.at[b])

    def _owt(i):
        off, sz = schunks[i]
        b = i % NB
        pltpu.make_async_copy(acc_sp.at[pl.ds(r0 + off, sz)],
                              rowsq.at[b, pl.ds(0, sz)], gsem.at[b]).wait()

    def _owr(i):
        off, sz = schunks[i]
        b = i % NB
        pltpu.async_copy(rowsq.at[b, pl.ds(0, sz)],
                         p_hbm.at[cid, pl.ds(r0 + off, sz)], ssem.at[b])

    def _odw(i):
        off, sz = schunks[i]
        b = i % NB
        pltpu.make_async_copy(rowsq.at[b, pl.ds(0, sz)],
                              p_hbm.at[cid, pl.ds(r0 + off, sz)],
                              ssem.at[b]).wait()

    for i in range(3):
        _ord(i)
    for i in range(3):
        _owt(i)
        _owr(i)
    for i in range(3, 5):
        _odw(i - 3)
        _ord(i)
    for i in range(3, 5):
        _owt(i)
        _owr(i)

    def _wbden(i, _):
        pltpu.sync_copy(den_sp.at[pl.ds(r0 + i * ZD, ZD)], zd_v)
        pltpu.sync_copy(zd_v, dn_hbm.at[pl.ds(cid * N_PAD + r0 + i * ZD, ZD)])
        return _
    lax.fori_loop(0, ROWS_PT // ZD, _wbden, None)

    @pl.when(sid == NS - 1)
    def _tail_out():
        t0 = N - ROWS_TAIL
        pltpu.sync_copy(acc_sp.at[pl.ds(t0, ROWS_TAIL)], rows_v)
        pltpu.sync_copy(rows_v, p_hbm.at[cid, pl.ds(t0, ROWS_TAIL)])
        pltpu.sync_copy(den_sp.at[pl.ds(t0, ROWS_TAIL)],
                        zd_v.at[pl.ds(0, ROWS_TAIL)])
        pltpu.sync_copy(zd_v.at[pl.ds(0, ROWS_TAIL)],
                        dn_hbm.at[pl.ds(cid * N_PAD + t0, ROWS_TAIL)])

    for i in range(2, 5):
        _odw(i)


_sc_edge = pl.kernel(
    _sc_edge_body,
    out_type=(
        jax.ShapeDtypeStruct((NC, N, D_HID), jnp.float32),
        jax.ShapeDtypeStruct((NC * N_PAD,), jnp.float32),
    ),
    mesh=plsc.VectorSubcoreMesh(
        core_axis_name="c", subcore_axis_name="s",
        num_cores=NC, num_subcores=NS),
    compiler_params=pltpu.CompilerParams(
        needs_layout_passes=False, use_tc_tiling_on_sc=False),
    scratch_types=[
        pltpu.VMEM_SHARED((N, D_HID), jnp.float32),
        pltpu.VMEM_SHARED((N, D_HID), jnp.float32),
        pltpu.VMEM_SHARED((N,), jnp.float32),
        pltpu.VMEM((N,), jnp.float32),
        pltpu.VMEM((N,), jnp.float32),
        pltpu.VMEM((NB, CH), jnp.int32),
        pltpu.VMEM((NB, CH), jnp.int32),
        pltpu.VMEM((NB, CH), jnp.float32),
        pltpu.VMEM((NB, CH, D_HID), jnp.float32),
        pltpu.VMEM((ROWS_TAIL, D_HID), jnp.float32),
        pltpu.VMEM((REM,), jnp.int32),
        pltpu.VMEM((REM,), jnp.int32),
        pltpu.VMEM((REM,), jnp.float32),
        pltpu.VMEM((REM, D_HID), jnp.float32),
        pltpu.VMEM((ZD,), jnp.float32),
        pltpu.SemaphoreType.DMA((NB,)),
        pltpu.SemaphoreType.DMA((NB,)),
        pltpu.SemaphoreType.DMA((NB,)),
    ],
)


def kernel(x, edge_index, W1, a1_src, a1_dst, W2, a2_src, a2_dst):
    src = edge_index[0]
    dst = edge_index[1]
    h1, as1, ad1 = _tc_embed(x, W1, a1_src, a1_dst)
    p1, d1f = _sc_edge(h1, src, dst, as1, ad1)
    z1 = _sc_comb(p1, d1f)
    h2, as2, ad2 = _tc_elu_embed(z1, W2, a2_src, a2_dst)
    p2, d2f = _sc_edge(h2, src, dst, as2, ad2)
    return _sc_comb(p2, d2f)
